# Initial kernel scaffold; baseline (speedup 1.0000x reference)
import jax
import jax.numpy as jnp
from jax.experimental import pallas as pl


def kernel(x, block_idx_tensor, params):
    raise NotImplementedError("write your pallas kernel here")


# speedup vs baseline: 1.4449x; 1.4449x over previous
import jax
import jax.numpy as jnp
from jax.experimental import pallas as pl
from jax.experimental.pallas import tpu as pltpu

B = 4096
IN = 1024
D = 512
NE = 8192
P = 4
ED = 64
HIDN = 128
ZD = P * ED
BETA = 0.25

BB = 512
NB = B // BB
R4 = BB * P
CC = 1024
NCH = NE // CC
HB = 1024
NHB = NE // HB


def _ln(x, g, b):
    m = jnp.mean(x, axis=-1, keepdims=True)
    v = jnp.mean((x - m) ** 2, axis=-1, keepdims=True)
    return (x - m) / jnp.sqrt(v + 1e-5) * g + b


def _fused(x_ref, bi_ref,
           encW, encb, resW, resb, resg, resbe,
           leT, ltT, wtT, wf1, wf2, wf3, wrc, fcb, ioW, iob,
           catWh, catWi, catb, cb3, cbT3,
           dinW, dinb, dresW, dresb, dresg, dresbe, doutW, doutb,
           xhat_ref, idx_ref, loss_ref, perp_ref,
           counts_ref, acc_ref):
    i = pl.program_id(0)
    f32 = jnp.float32

    @pl.when(i == 0)
    def _init():
        counts_ref[...] = jnp.zeros_like(counts_ref)
        acc_ref[0, 0] = 0.0

    def dot(a, bm):
        return jax.lax.dot(a, bm, precision=jax.lax.Precision.HIGHEST,
                           preferred_element_type=f32)

    h = dot(x_ref[...], encW[...]) + encb[...]
    for _ in range(2):
        t = dot(h, resW[...]) + resb[...]
        h = h + jax.nn.relu(_ln(t, resg[...], resbe[...]))

    bi = bi_ref[...]
    oh_le = (bi[:, 0:1] == jax.lax.broadcasted_iota(jnp.int32, (1, 32), 1)).astype(f32)
    oh_lt = (bi[:, 1:2] == jax.lax.broadcasted_iota(jnp.int32, (1, 8), 1)).astype(f32)
    oh_wt = (bi[:, 2:3] == jax.lax.broadcasted_iota(jnp.int32, (1, 8), 1)).astype(f32)
    hid = dot(dot(oh_le, leT[...]), wf1[...])
    hid = hid + dot(dot(oh_lt, ltT[...]), wf2[...])
    hid = hid + dot(dot(oh_wt, wtT[...]), wf3[...])
    hid = hid + bi[:, 3:4].astype(f32) * wrc[0:1, :]
    hid = hid + bi[:, 4:5].astype(f32) * wrc[1:2, :]
    hid = hid + bi[:, 5:6].astype(f32) * wrc[2:3, :]
    hid = hid + fcb[...]
    out_idx = dot(hid, ioW[...]) + iob[...]

    z_e = dot(h, catWh[...]) + dot(out_idx, catWi[...]) + catb[...]

    z4 = jnp.concatenate([z_e[:, p * ED:(p + 1) * ED] for p in range(P)], axis=0)
    zsq = jnp.sum(z4 * z4, axis=1, keepdims=True)

    def chunk(c, carry):
        bd, bix, bz = carry
        ect = cbT3[c, :, :]
        ec = cb3[c, :, :]
        cbsq = jnp.sum(ect * ect, axis=0, keepdims=True)
        dmat = (zsq + cbsq) - 2.0 * dot(z4, ect)
        md = jnp.min(dmat, axis=1, keepdims=True)
        iota = jax.lax.broadcasted_iota(jnp.int32, (R4, CC), 1)
        la = jnp.min(jnp.where(dmat == md, iota, NE), axis=1, keepdims=True)
        oh = (iota == la).astype(f32)
        zsel = dot(oh, ec)
        better = md < bd
        bd = jnp.where(better, md, bd)
        bix = jnp.where(better, la + c * CC, bix)
        bz = jnp.where(better, zsel, bz)
        return bd, bix, bz

    bd0 = jnp.full((R4, 1), jnp.inf, f32)
    bix0 = jnp.zeros((R4, 1), jnp.int32)
    bz0 = jnp.zeros((R4, ED), f32)
    _, bix, bz = jax.lax.fori_loop(0, NCH, chunk, (bd0, bix0, bz0))

    z_q = jnp.concatenate([bz[p * BB:(p + 1) * BB, :] for p in range(P)], axis=1)
    idx_ref[...] = jnp.concatenate([bix[p * BB:(p + 1) * BB, :] for p in range(P)], axis=1)

    acc_ref[0, 0] += jnp.sum((z_q - z_e) ** 2)

    for hc in range(NHB):
        iota_h = jax.lax.broadcasted_iota(jnp.int32, (1, HB), 1) + hc * HB
        cnt = jnp.sum((bix == iota_h).astype(f32), axis=0, keepdims=True)
        counts_ref[hc:hc + 1, :] += cnt

    dh = dot(z_q, dinW[...]) + dinb[...]
    for _ in range(2):
        t = dot(dh, dresW[...]) + dresb[...]
        dh = dh + jax.nn.relu(_ln(t, dresg[...], dresbe[...]))
    xhat_ref[...] = dot(dh, doutW[...]) + doutb[...]

    @pl.when(i == NB - 1)
    def _fin():
        loss_ref[...] = jnp.reshape(acc_ref[0, 0] * ((1.0 + BETA) / (B * ZD)), (1, 1))
        e = counts_ref[...] * (1.0 / (B * P))
        ent = jnp.sum(e * jnp.log(e + 1e-10))
        perp_ref[...] = jnp.reshape(jnp.exp(-ent), (1, 1))


def _cspec(shape):
    n = len(shape)
    return pl.BlockSpec(shape, lambda i, _n=n: (0,) * _n)


@jax.jit
def kernel(x, block_idx_tensor, params):
    p = params
    f32 = jnp.float32

    def row(v):
        return v.reshape(1, -1)

    ltT = jnp.zeros((8, HIDN), f32).at[:2].set(p['ltype_emb'])
    wtT = jnp.zeros((8, HIDN), f32).at[:7].set(p['wtype_emb'])
    fcW = p['idx_fc_W']
    wf1, wf2, wf3 = fcW[0:128], fcW[128:256], fcW[256:384]
    wrc = jnp.zeros((8, HIDN), f32).at[:3].set(fcW[384:387])
    catWh, catWi = p['cat_W'][:D], p['cat_W'][D:]
    cb = p['codebook']
    cb3 = cb.reshape(NCH, CC, ED)
    cbT3 = cb.T.reshape(ED, NCH, CC).transpose(1, 0, 2)
    bi = block_idx_tensor.astype(jnp.int32)

    xhat, idx2, lossv, perpv = pl.pallas_call(
        _fused,
        grid=(NB,),
        in_specs=[
            pl.BlockSpec((BB, IN), lambda i: (i, 0)),
            pl.BlockSpec((BB, 6), lambda i: (i, 0)),
            _cspec((IN, D)), _cspec((1, D)),
            _cspec((D, D)), _cspec((1, D)), _cspec((1, D)), _cspec((1, D)),
            _cspec((32, HIDN)), _cspec((8, HIDN)), _cspec((8, HIDN)),
            _cspec((HIDN, HIDN)), _cspec((HIDN, HIDN)), _cspec((HIDN, HIDN)),
            _cspec((8, HIDN)), _cspec((1, HIDN)),
            _cspec((HIDN, ED)), _cspec((1, ED)),
            _cspec((D, ZD)), _cspec((ED, ZD)), _cspec((1, ZD)),
            _cspec((NCH, CC, ED)), _cspec((NCH, ED, CC)),
            _cspec((ZD, D)), _cspec((1, D)),
            _cspec((D, D)), _cspec((1, D)), _cspec((1, D)), _cspec((1, D)),
            _cspec((D, IN)), _cspec((1, IN)),
        ],
        out_specs=[
            pl.BlockSpec((BB, IN), lambda i: (i, 0)),
            pl.BlockSpec((BB, P), lambda i: (i, 0)),
            pl.BlockSpec((1, 1), lambda i: (0, 0)),
            pl.BlockSpec((1, 1), lambda i: (0, 0)),
        ],
        out_shape=[
            jax.ShapeDtypeStruct((B, IN), f32),
            jax.ShapeDtypeStruct((B, P), jnp.int32),
            jax.ShapeDtypeStruct((1, 1), f32),
            jax.ShapeDtypeStruct((1, 1), f32),
        ],
        scratch_shapes=[
            pltpu.VMEM((NHB, HB), f32),
            pltpu.SMEM((1, 1), f32),
        ],
    )(x, bi,
      p['enc_W'], row(p['enc_b']),
      p['enc_res_W'], row(p['enc_res_b']), row(p['enc_res_g']), row(p['enc_res_be']),
      p['layer_emb'], ltT, wtT, wf1, wf2, wf3, wrc, row(p['idx_fc_b']),
      p['idx_out_W'], row(p['idx_out_b']),
      catWh, catWi, row(p['cat_b']),
      cb3, cbT3,
      p['dec_in_W'], row(p['dec_in_b']),
      p['dec_res_W'], row(p['dec_res_b']), row(p['dec_res_g']), row(p['dec_res_be']),
      p['dec_out_W'], row(p['dec_out_b']))
    return lossv[0, 0], xhat, perpv[0, 0], idx2.reshape(-1)


# speedup vs baseline: 3.7348x; 2.5849x over previous
import jax
import jax.numpy as jnp
from jax.experimental import pallas as pl
from jax.experimental.pallas import tpu as pltpu

B = 4096
IN = 1024
D = 512
NE = 8192
P = 4
ED = 64
HIDN = 128
ZD = P * ED
BETA = 0.25

BB = 512
NB = B // BB
R4 = BB * P
CC = 1024
NCH = NE // CC
HB = 1024
NHB = NE // HB


def _ln(x, g, b):
    m = jnp.mean(x, axis=-1, keepdims=True)
    v = jnp.mean((x - m) ** 2, axis=-1, keepdims=True)
    return (x - m) / jnp.sqrt(v + 1e-5) * g + b


def _fused(x_ref, bi_ref,
           encW, encb, resW, resb, resg, resbe,
           leT, ltT, wtT, wf1, wf2, wf3, wrc, fcb, ioW, iob,
           catWh, catWi, catb, cb3, cbT3,
           dinW, dinb, dresW, dresb, dresg, dresbe, doutW, doutb,
           xhat_ref, idx_ref, loss_ref, perp_ref,
           counts_ref, acc_ref):
    i = pl.program_id(0)
    f32 = jnp.float32

    @pl.when(i == 0)
    def _init():
        counts_ref[...] = jnp.zeros_like(counts_ref)
        acc_ref[0, 0] = 0.0

    def dot(a, bm):
        return jax.lax.dot(a, bm, preferred_element_type=f32)

    def dot_hi(a, bm):
        return jax.lax.dot(a, bm, precision=jax.lax.Precision.HIGHEST,
                           preferred_element_type=f32)

    h = dot(x_ref[...], encW[...]) + encb[...]
    for _ in range(2):
        t = dot(h, resW[...]) + resb[...]
        h = h + jax.nn.relu(_ln(t, resg[...], resbe[...]))

    bi = bi_ref[...]
    oh_le = (bi[:, 0:1] == jax.lax.broadcasted_iota(jnp.int32, (1, 32), 1)).astype(f32)
    oh_lt = (bi[:, 1:2] == jax.lax.broadcasted_iota(jnp.int32, (1, 8), 1)).astype(f32)
    oh_wt = (bi[:, 2:3] == jax.lax.broadcasted_iota(jnp.int32, (1, 8), 1)).astype(f32)
    hid = dot(dot(oh_le, leT[...]), wf1[...])
    hid = hid + dot(dot(oh_lt, ltT[...]), wf2[...])
    hid = hid + dot(dot(oh_wt, wtT[...]), wf3[...])
    hid = hid + bi[:, 3:4].astype(f32) * wrc[0:1, :]
    hid = hid + bi[:, 4:5].astype(f32) * wrc[1:2, :]
    hid = hid + bi[:, 5:6].astype(f32) * wrc[2:3, :]
    hid = hid + fcb[...]
    out_idx = dot(hid, ioW[...]) + iob[...]

    z_e = dot(h, catWh[...]) + dot(out_idx, catWi[...]) + catb[...]

    z4 = jnp.concatenate([z_e[:, p * ED:(p + 1) * ED] for p in range(P)], axis=0)
    zsq = jnp.sum(z4 * z4, axis=1, keepdims=True)

    def chunk(c, carry):
        bd, bix, bz = carry
        ect = cbT3[c, :, :]
        ec = cb3[c, :, :]
        cbsq = jnp.sum(ect * ect, axis=0, keepdims=True)
        dmat = (zsq + cbsq) - 2.0 * dot_hi(z4, ect)
        md = jnp.min(dmat, axis=1, keepdims=True)
        iota = jax.lax.broadcasted_iota(jnp.int32, (R4, CC), 1)
        la = jnp.min(jnp.where(dmat == md, iota, NE), axis=1, keepdims=True)
        oh = (iota == la).astype(f32)
        zsel = dot(oh, ec)
        better = md < bd
        bd = jnp.where(better, md, bd)
        bix = jnp.where(better, la + c * CC, bix)
        bz = jnp.where(better, zsel, bz)
        return bd, bix, bz

    bd0 = jnp.full((R4, 1), jnp.inf, f32)
    bix0 = jnp.zeros((R4, 1), jnp.int32)
    bz0 = jnp.zeros((R4, ED), f32)
    _, bix, bz = jax.lax.fori_loop(0, NCH, chunk, (bd0, bix0, bz0))

    z_q = jnp.concatenate([bz[p * BB:(p + 1) * BB, :] for p in range(P)], axis=1)
    idx_ref[...] = jnp.concatenate([bix[p * BB:(p + 1) * BB, :] for p in range(P)], axis=1)

    acc_ref[0, 0] += jnp.sum((z_q - z_e) ** 2)

    for hc in range(NHB):
        iota_h = jax.lax.broadcasted_iota(jnp.int32, (1, HB), 1) + hc * HB
        cnt = jnp.sum((bix == iota_h).astype(f32), axis=0, keepdims=True)
        counts_ref[hc:hc + 1, :] += cnt

    dh = dot(z_q, dinW[...]) + dinb[...]
    for _ in range(2):
        t = dot(dh, dresW[...]) + dresb[...]
        dh = dh + jax.nn.relu(_ln(t, dresg[...], dresbe[...]))
    xhat_ref[...] = dot(dh, doutW[...]) + doutb[...]

    @pl.when(i == NB - 1)
    def _fin():
        loss_ref[...] = jnp.reshape(acc_ref[0, 0] * ((1.0 + BETA) / (B * ZD)), (1, 1))
        e = counts_ref[...] * (1.0 / (B * P))
        ent = jnp.sum(e * jnp.log(e + 1e-10))
        perp_ref[...] = jnp.reshape(jnp.exp(-ent), (1, 1))


def _cspec(shape):
    n = len(shape)
    return pl.BlockSpec(shape, lambda i, _n=n: (0,) * _n)


@jax.jit
def kernel(x, block_idx_tensor, params):
    p = params
    f32 = jnp.float32

    def row(v):
        return v.reshape(1, -1)

    ltT = jnp.zeros((8, HIDN), f32).at[:2].set(p['ltype_emb'])
    wtT = jnp.zeros((8, HIDN), f32).at[:7].set(p['wtype_emb'])
    fcW = p['idx_fc_W']
    wf1, wf2, wf3 = fcW[0:128], fcW[128:256], fcW[256:384]
    wrc = jnp.zeros((8, HIDN), f32).at[:3].set(fcW[384:387])
    catWh, catWi = p['cat_W'][:D], p['cat_W'][D:]
    cb = p['codebook']
    cb3 = cb.reshape(NCH, CC, ED)
    cbT3 = cb.T.reshape(ED, NCH, CC).transpose(1, 0, 2)
    bi = block_idx_tensor.astype(jnp.int32)

    xhat, idx2, lossv, perpv = pl.pallas_call(
        _fused,
        grid=(NB,),
        in_specs=[
            pl.BlockSpec((BB, IN), lambda i: (i, 0)),
            pl.BlockSpec((BB, 6), lambda i: (i, 0)),
            _cspec((IN, D)), _cspec((1, D)),
            _cspec((D, D)), _cspec((1, D)), _cspec((1, D)), _cspec((1, D)),
            _cspec((32, HIDN)), _cspec((8, HIDN)), _cspec((8, HIDN)),
            _cspec((HIDN, HIDN)), _cspec((HIDN, HIDN)), _cspec((HIDN, HIDN)),
            _cspec((8, HIDN)), _cspec((1, HIDN)),
            _cspec((HIDN, ED)), _cspec((1, ED)),
            _cspec((D, ZD)), _cspec((ED, ZD)), _cspec((1, ZD)),
            _cspec((NCH, CC, ED)), _cspec((NCH, ED, CC)),
            _cspec((ZD, D)), _cspec((1, D)),
            _cspec((D, D)), _cspec((1, D)), _cspec((1, D)), _cspec((1, D)),
            _cspec((D, IN)), _cspec((1, IN)),
        ],
        out_specs=[
            pl.BlockSpec((BB, IN), lambda i: (i, 0)),
            pl.BlockSpec((BB, P), lambda i: (i, 0)),
            pl.BlockSpec((1, 1), lambda i: (0, 0)),
            pl.BlockSpec((1, 1), lambda i: (0, 0)),
        ],
        out_shape=[
            jax.ShapeDtypeStruct((B, IN), f32),
            jax.ShapeDtypeStruct((B, P), jnp.int32),
            jax.ShapeDtypeStruct((1, 1), f32),
            jax.ShapeDtypeStruct((1, 1), f32),
        ],
        scratch_shapes=[
            pltpu.VMEM((NHB, HB), f32),
            pltpu.SMEM((1, 1), f32),
        ],
    )(x, bi,
      p['enc_W'], row(p['enc_b']),
      p['enc_res_W'], row(p['enc_res_b']), row(p['enc_res_g']), row(p['enc_res_be']),
      p['layer_emb'], ltT, wtT, wf1, wf2, wf3, wrc, row(p['idx_fc_b']),
      p['idx_out_W'], row(p['idx_out_b']),
      catWh, catWi, row(p['cat_b']),
      cb3, cbT3,
      p['dec_in_W'], row(p['dec_in_b']),
      p['dec_res_W'], row(p['dec_res_b']), row(p['dec_res_g']), row(p['dec_res_be']),
      p['dec_out_W'], row(p['dec_out_b']))
    return lossv[0, 0], xhat, perpv[0, 0], idx2.reshape(-1)
